# baseline (speedup 1.0000x reference)
import jax
import jax.numpy as jnp
from jax import lax
from jax.experimental import pallas as pl
from jax.experimental.pallas import tpu as pltpu
from jax.experimental.pallas import tpu_sc as plsc

N_ATOMS = 100000
D = 128
ZMAX = 87

NC = 2
NS = 16
NW = NC * NS

CHUNK = 128
N_CHUNKS = (N_ATOMS + CHUNK - 1) // CHUNK
MAX_PER_WORKER = (N_CHUNKS + NW - 1) // NW


def _table_body(emb_ref, ec_ref, w_ref, out_ref):
    out_ref[...] = emb_ref[...] + lax.dot_general(
        ec_ref[...], w_ref[...],
        dimension_numbers=(((1,), (1,)), ((), ())),
        preferred_element_type=jnp.float32,
    )


def _build_table(element_embedding, W, electron_config):
    return pl.pallas_call(
        _table_body,
        out_shape=jax.ShapeDtypeStruct((ZMAX, D), jnp.float32),
    )(element_embedding, electron_config, W)


IDX_PER_W = MAX_PER_WORKER * CHUNK
NBUF = 4


def _gather_body(table_hbm, z_hbm, out_hbm, idx_v, rows_v, gsem, wsem):
    wid = lax.axis_index("s") * NC + lax.axis_index("c")
    base_w = jnp.minimum(wid * IDX_PER_W, N_ATOMS - IDX_PER_W)
    pltpu.sync_copy(z_hbm.at[pl.ds(base_w, IDX_PER_W)], idx_v)

    def gather(j):
        b = j % NBUF
        return pltpu.async_copy(
            table_hbm.at[idx_v.at[pl.ds(j * CHUNK, CHUNK)]],
            rows_v.at[b], gsem.at[b])

    gd = [None] * MAX_PER_WORKER
    wd = [None] * MAX_PER_WORKER
    for j in range(NBUF):
        gd[j] = gather(j)
    for j in range(MAX_PER_WORKER):
        b = j % NBUF
        gd[j].wait()
        wd[j] = pltpu.async_copy(
            rows_v.at[b], out_hbm.at[pl.ds(base_w + j * CHUNK, CHUNK)],
            wsem.at[b])
        if j >= 1 and (j - 1 + NBUF) < MAX_PER_WORKER:
            wd[j - 1].wait()
            gd[j - 1 + NBUF] = gather(j - 1 + NBUF)
    for j in range(max(0, MAX_PER_WORKER - NBUF), MAX_PER_WORKER):
        wd[j].wait()


_gather = pl.kernel(
    _gather_body,
    out_type=jax.ShapeDtypeStruct((N_ATOMS, D), jnp.float32),
    mesh=plsc.VectorSubcoreMesh(core_axis_name="c", subcore_axis_name="s"),
    scratch_types=[
        pltpu.VMEM((IDX_PER_W,), jnp.int32),
        pltpu.VMEM((NBUF, CHUNK, D), jnp.float32),
        pltpu.SemaphoreType.DMA((NBUF,)),
        pltpu.SemaphoreType.DMA((NBUF,)),
    ],
)


def kernel(Z, element_embedding, W, electron_config):
    table = _build_table(element_embedding, W, electron_config)
    return _gather(table, Z.astype(jnp.int32))


# speedup vs baseline: 3.9357x; 3.9357x over previous
import jax
import jax.numpy as jnp
from jax import lax
from jax.experimental import pallas as pl
from jax.experimental.pallas import tpu as pltpu
from jax.experimental.pallas import tpu_sc as plsc

N_ATOMS = 100000
D = 128
ZMAX = 87

NC = 2
NS = 16
NW = NC * NS

CHUNK = 128
N_CHUNKS = (N_ATOMS + CHUNK - 1) // CHUNK
MAX_PER_WORKER = (N_CHUNKS + NW - 1) // NW


def _table_body(emb_ref, ec_ref, w_ref, out_ref):
    out_ref[...] = emb_ref[...] + lax.dot_general(
        ec_ref[...], w_ref[...],
        dimension_numbers=(((1,), (1,)), ((), ())),
        preferred_element_type=jnp.float32,
    )


def _build_table(element_embedding, W, electron_config):
    return pl.pallas_call(
        _table_body,
        out_shape=jax.ShapeDtypeStruct((ZMAX, D), jnp.float32),
    )(element_embedding, electron_config, W)


IDX_PER_W = MAX_PER_WORKER * CHUNK
NBUF = 4


def _gather_body(table_hbm, z_hbm, out_hbm, idx_v, rows_v, stage_v, table_sh,
                 gsem, wsem):
    sid = lax.axis_index("s")
    wid = sid * NC + lax.axis_index("c")
    base_w = jnp.minimum(wid * IDX_PER_W, N_ATOMS - IDX_PER_W)
    pltpu.sync_copy(z_hbm.at[pl.ds(base_w, IDX_PER_W)], idx_v)

    @pl.when(sid == 0)
    def _():
        pltpu.sync_copy(table_hbm, stage_v)
        pltpu.sync_copy(stage_v, table_sh)

    plsc.subcore_barrier()

    def gather(j):
        b = j % NBUF
        return pltpu.async_copy(
            table_sh.at[idx_v.at[pl.ds(j * CHUNK, CHUNK)]],
            rows_v.at[b], gsem.at[b])

    gd = [None] * MAX_PER_WORKER
    wd = [None] * MAX_PER_WORKER
    for j in range(NBUF):
        gd[j] = gather(j)
    for j in range(MAX_PER_WORKER):
        b = j % NBUF
        gd[j].wait()
        wd[j] = pltpu.async_copy(
            rows_v.at[b], out_hbm.at[pl.ds(base_w + j * CHUNK, CHUNK)],
            wsem.at[b])
        if j >= 1 and (j - 1 + NBUF) < MAX_PER_WORKER:
            wd[j - 1].wait()
            gd[j - 1 + NBUF] = gather(j - 1 + NBUF)
    for j in range(max(0, MAX_PER_WORKER - NBUF), MAX_PER_WORKER):
        wd[j].wait()


_gather = pl.kernel(
    _gather_body,
    out_type=jax.ShapeDtypeStruct((N_ATOMS, D), jnp.float32),
    mesh=plsc.VectorSubcoreMesh(core_axis_name="c", subcore_axis_name="s"),
    scratch_types=[
        pltpu.VMEM((IDX_PER_W,), jnp.int32),
        pltpu.VMEM((NBUF, CHUNK, D), jnp.float32),
        pltpu.VMEM((ZMAX, D), jnp.float32),
        pltpu.VMEM_SHARED((ZMAX, D), jnp.float32),
        pltpu.SemaphoreType.DMA((NBUF,)),
        pltpu.SemaphoreType.DMA((NBUF,)),
    ],
)


def kernel(Z, element_embedding, W, electron_config):
    table = _build_table(element_embedding, W, electron_config)
    return _gather(table, Z.astype(jnp.int32))
